# baseline (speedup 1.0000x reference)
import dataclasses
import functools

import jax
import jax.numpy as jnp
from jax import lax
from jax.experimental import pallas as pl
from jax.experimental.pallas import tpu as pltpu
from jax.experimental.pallas import tpu_sc as plsc

BATCH = 16384
LATENT_DIM = 32
NUM_CORES = 2
NUM_SUBCORES = 16
NUM_WORKERS = NUM_CORES * NUM_SUBCORES
B_PER_W = BATCH // NUM_WORKERS
LANES = 16
CHUNK = 128
NCHUNKS = B_PER_W // CHUNK
GROUPS_PER_CHUNK = CHUNK // LANES


def _mf_kernel_body(uidx_hbm, sidx_hbm, ut_hbm, st_hbm, ub_hbm, sb_hbm,
                    out_hbm, idx_u, idx_s, row_u, row_s, cb_u, cb_s,
                    ubuf0, ubuf1, sbuf0, sbuf1, ub_el, sb_el, out_v,
                    semu0, semu1, sems0, sems1, semb0, semb1):
    wid = lax.axis_index("s") * NUM_CORES + lax.axis_index("c")
    base = wid * B_PER_W

    pltpu.sync_copy(uidx_hbm.at[pl.ds(base, B_PER_W)], idx_u)
    pltpu.sync_copy(sidx_hbm.at[pl.ds(base, B_PER_W)], idx_s)

    @pl.loop(0, B_PER_W // LANES)
    def _(g):
        sl = pl.ds(g * LANES, LANES)
        iu = idx_u[sl]
        isg = idx_s[sl]
        row_u[sl] = lax.shift_right_logical(iu, 2)
        row_s[sl] = lax.shift_right_logical(isg, 2)
        cb_u[sl] = (iu & 3) * 32
        cb_s[sl] = (isg & 3) * 32

    cpb0 = pltpu.async_copy(ub_hbm.at[idx_u], ub_el, semb0)
    cpb1 = pltpu.async_copy(sb_hbm.at[idx_s], sb_el, semb1)

    ubufs = (ubuf0, ubuf1)
    sbufs = (sbuf0, sbuf1)
    usems = (semu0, semu1)
    ssems = (sems0, sems1)

    def issue(c):
        rs = pl.ds(c * CHUNK, CHUNK)
        cu = pltpu.async_copy(ut_hbm.at[row_u.at[rs]], ubufs[c % 2],
                              usems[c % 2])
        cs = pltpu.async_copy(st_hbm.at[row_s.at[rs]], sbufs[c % 2],
                              ssems[c % 2])
        return cu, cs

    lane = lax.iota(jnp.int32, LANES)
    pend = issue(0)
    cpb0.wait()
    cpb1.wait()

    for c in range(NCHUNKS):
        nxt = issue(c + 1) if c + 1 < NCHUNKS else None
        pend[0].wait()
        pend[1].wait()
        ubuf = ubufs[c % 2]
        sbuf = sbufs[c % 2]

        @pl.loop(0, GROUPS_PER_CHUNK)
        def _(g):
            gl = c * CHUNK + g * LANES
            sl = pl.ds(gl, LANES)
            rows16 = lane + g * LANES
            cbu = cb_u[sl]
            cbs = cb_s[sl]
            acc = ub_el[sl] + sb_el[sl]
            for d in range(LATENT_DIM):
                dvec = (lane + d) & (LATENT_DIM - 1)
                gu = plsc.load_gather(ubuf, [rows16, cbu + dvec])
                gs = plsc.load_gather(sbuf, [rows16, cbs + dvec])
                acc = acc + gu * gs
            out_v[sl] = acc

        pend = nxt

    pltpu.sync_copy(out_v, out_hbm.at[pl.ds(base, B_PER_W)])


@jax.jit
def kernel(user_idxs, song_idxs, user_table, song_table, user_bias,
           song_bias):
    user_idxs = user_idxs.astype(jnp.int32)
    song_idxs = song_idxs.astype(jnp.int32)
    ut_v = user_table.reshape(-1, 128)
    st_v = song_table.reshape(-1, 128)
    ub_f = user_bias.reshape(-1)
    sb_f = song_bias.reshape(-1)

    mesh = plsc.VectorSubcoreMesh(core_axis_name="c", subcore_axis_name="s")
    cp = pltpu.CompilerParams()
    fields = pltpu.CompilerParams.__dataclass_fields__
    if "needs_layout_passes" in fields:
        cp = dataclasses.replace(cp, needs_layout_passes=False)
    f = functools.partial(
        pl.kernel,
        compiler_params=cp,
        out_type=jax.ShapeDtypeStruct((BATCH,), jnp.float32),
        mesh=mesh,
        scratch_types=[
            pltpu.VMEM((B_PER_W,), jnp.int32),
            pltpu.VMEM((B_PER_W,), jnp.int32),
            pltpu.VMEM((B_PER_W,), jnp.int32),
            pltpu.VMEM((B_PER_W,), jnp.int32),
            pltpu.VMEM((B_PER_W,), jnp.int32),
            pltpu.VMEM((B_PER_W,), jnp.int32),
            pltpu.VMEM((CHUNK, 128), jnp.float32),
            pltpu.VMEM((CHUNK, 128), jnp.float32),
            pltpu.VMEM((CHUNK, 128), jnp.float32),
            pltpu.VMEM((CHUNK, 128), jnp.float32),
            pltpu.VMEM((B_PER_W,), jnp.float32),
            pltpu.VMEM((B_PER_W,), jnp.float32),
            pltpu.VMEM((B_PER_W,), jnp.float32),
            pltpu.SemaphoreType.DMA,
            pltpu.SemaphoreType.DMA,
            pltpu.SemaphoreType.DMA,
            pltpu.SemaphoreType.DMA,
            pltpu.SemaphoreType.DMA,
            pltpu.SemaphoreType.DMA,
        ],
    )(_mf_kernel_body)
    return f(user_idxs, song_idxs, ut_v, st_v, ub_f, sb_f)


# speedup vs baseline: 1.0074x; 1.0074x over previous
import dataclasses
import functools

import jax
import jax.numpy as jnp
from jax import lax
from jax.experimental import pallas as pl
from jax.experimental.pallas import tpu as pltpu
from jax.experimental.pallas import tpu_sc as plsc

BATCH = 16384
LATENT_DIM = 32
NUM_CORES = 2
NUM_SUBCORES = 16
NUM_WORKERS = NUM_CORES * NUM_SUBCORES
B_PER_W = BATCH // NUM_WORKERS
LANES = 16


def _mf_kernel_body(uidx_hbm, sidx_hbm, ut_hbm, st_hbm, ub_hbm, sb_hbm,
                    out_hbm, idx_u, idx_s, idx_hi_u, idx_hi_s, u_rows,
                    s_rows, ub_rows, sb_rows, out_v, sem0, sem1, sem2,
                    sem3):
    wid = lax.axis_index("s") * NUM_CORES + lax.axis_index("c")
    base = wid * B_PER_W

    pltpu.sync_copy(uidx_hbm.at[pl.ds(base, B_PER_W)], idx_u)
    pltpu.sync_copy(sidx_hbm.at[pl.ds(base, B_PER_W)], idx_s)

    @pl.loop(0, B_PER_W // LANES)
    def _(g):
        sl = pl.ds(g * LANES, LANES)
        idx_hi_u[sl] = lax.shift_right_logical(idx_u[sl], 4)
        idx_hi_s[sl] = lax.shift_right_logical(idx_s[sl], 4)

    cp0 = pltpu.async_copy(ut_hbm.at[idx_u], u_rows, sem0)
    cp1 = pltpu.async_copy(st_hbm.at[idx_s], s_rows, sem1)
    cp2 = pltpu.async_copy(ub_hbm.at[idx_hi_u], ub_rows, sem2)
    cp3 = pltpu.async_copy(sb_hbm.at[idx_hi_s], sb_rows, sem3)
    cp0.wait()
    cp1.wait()
    cp2.wait()
    cp3.wait()

    lane = lax.iota(jnp.int32, LANES)

    @pl.loop(0, B_PER_W // LANES)
    def _(g):
        gbase = g * LANES
        sl = pl.ds(gbase, LANES)
        rows16 = lane + gbase
        ubias = plsc.load_gather(ub_rows, [rows16, idx_u[sl] & 15])
        sbias = plsc.load_gather(sb_rows, [rows16, idx_s[sl] & 15])
        out_vec = jnp.zeros((LANES,), jnp.float32)
        for j in range(LANES):
            r = gbase + j
            prod = (u_rows[r, pl.ds(0, LANES)] * s_rows[r, pl.ds(0, LANES)]
                    + u_rows[r, pl.ds(LANES, LANES)]
                    * s_rows[r, pl.ds(LANES, LANES)])
            tot = jnp.sum(prod)
            out_vec = jnp.where(lane == j, tot, out_vec)
        out_v[sl] = out_vec + ubias + sbias

    pltpu.sync_copy(out_v, out_hbm.at[pl.ds(base, B_PER_W)])


@jax.jit
def kernel(user_idxs, song_idxs, user_table, song_table, user_bias,
           song_bias):
    user_idxs = user_idxs.astype(jnp.int32)
    song_idxs = song_idxs.astype(jnp.int32)
    ub_g = user_bias.reshape(-1, LANES)
    sb_g = song_bias.reshape(-1, LANES)

    mesh = plsc.VectorSubcoreMesh(core_axis_name="c", subcore_axis_name="s")
    cp = pltpu.CompilerParams()
    fields = pltpu.CompilerParams.__dataclass_fields__
    if "needs_layout_passes" in fields:
        cp = dataclasses.replace(cp, needs_layout_passes=False)
    if "use_tc_tiling_on_sc" in fields:
        cp = dataclasses.replace(cp, use_tc_tiling_on_sc=False)
    f = functools.partial(
        pl.kernel,
        compiler_params=cp,
        out_type=jax.ShapeDtypeStruct((BATCH,), jnp.float32),
        mesh=mesh,
        scratch_types=[
            pltpu.VMEM((B_PER_W,), jnp.int32),
            pltpu.VMEM((B_PER_W,), jnp.int32),
            pltpu.VMEM((B_PER_W,), jnp.int32),
            pltpu.VMEM((B_PER_W,), jnp.int32),
            pltpu.VMEM((B_PER_W, LATENT_DIM), jnp.float32),
            pltpu.VMEM((B_PER_W, LATENT_DIM), jnp.float32),
            pltpu.VMEM((B_PER_W, LANES), jnp.float32),
            pltpu.VMEM((B_PER_W, LANES), jnp.float32),
            pltpu.VMEM((B_PER_W,), jnp.float32),
            pltpu.SemaphoreType.DMA,
            pltpu.SemaphoreType.DMA,
            pltpu.SemaphoreType.DMA,
            pltpu.SemaphoreType.DMA,
        ],
    )(_mf_kernel_body)
    return f(user_idxs, song_idxs, user_table, song_table, ub_g, sb_g)
